# baseline (speedup 1.0000x reference)
import functools

import jax
import jax.numpy as jnp
from jax import lax
from jax.experimental import pallas as pl
from jax.experimental.pallas import tpu as pltpu
from jax.experimental.pallas import tpu_sc as plsc

N_NODES = 10000
N_EDGES = 160000
N_PAD = 10240
NSC = 2
NTILES = 16
E_PAD = 160256
HALF_E = E_PAD // NSC
NPIECE = 16
PC = HALF_E // NPIECE
NGRP = PC // 16
PCAP = PC + 16


def _make_sc_agg(d, with_deg):
    ch = 256 if d == 256 else 128
    g = 80 if d == 256 else 48
    nch = -(-N_PAD // ch)
    npass = -(-nch // NTILES)

    outs = [jax.ShapeDtypeStruct((NSC * N_PAD, d), jnp.float32)]
    if with_deg:
        outs.append(jax.ShapeDtypeStruct((NSC * N_PAD,), jnp.float32))

    scratch = [
        pltpu.VMEM((PC,), jnp.int32),
        pltpu.VMEM((PC,), jnp.int32),
        pltpu.VMEM((PCAP,), jnp.int32),
        pltpu.VMEM((PCAP,), jnp.int32),
        pltpu.VMEM((g, d), jnp.float32),
        pltpu.VMEM((g,), jnp.int32),
        pltpu.VMEM((ch, d), jnp.float32),
        pltpu.SemaphoreType.DMA,
    ]
    if with_deg:
        scratch += [pltpu.VMEM((ch,), jnp.float32)]

    def body(x_hbm, src_hbm, dst_hbm, *refs):
        if with_deg:
            (agg_hbm, deg_hbm, srcp_v, dstp_v, pend_s, pend_d,
             rows_v, isrc_v, acc_v, sem, deg_v) = refs
        else:
            (agg_hbm, srcp_v, dstp_v, pend_s, pend_d,
             rows_v, isrc_v, acc_v, sem) = refs

        c = lax.axis_index("c")
        s = lax.axis_index("s")
        cbase = c * N_PAD
        z16 = jnp.zeros((16,), jnp.float32)
        zi16 = jnp.zeros((16,), jnp.int32)
        ones16 = jnp.full((16,), 1.0, jnp.float32)
        iota16 = lax.iota(jnp.int32, 16)

        def pzbody(i, _):
            pend_s[pl.ds(i * 16, 16)] = zi16
            pend_d[pl.ds(i * 16, 16)] = zi16
            return 0
        lax.fori_loop(0, PCAP // 16, pzbody, 0)

        for ps in range(npass):
            q = ps * NTILES + s
            lo = q * ch

            def azbody(r, _):
                for cc in range(d // 16):
                    acc_v[r, pl.ds(cc * 16, 16)] = z16
                return 0
            lax.fori_loop(0, ch, azbody, 0)
            if with_deg:
                def dzbody(i, _):
                    deg_v[pl.ds(i * 16, 16)] = z16
                    return 0
                lax.fori_loop(0, ch // 16, dzbody, 0)

            def piece(p, _):
                ebase = c * HALF_E + p * PC
                pltpu.sync_copy(src_hbm.at[pl.ds(ebase, PC)], srcp_v)
                pltpu.sync_copy(dst_hbm.at[pl.ds(ebase, PC)], dstp_v)

                def scan_g(gg, cnt):
                    d16 = dstp_v[pl.ds(gg * 16, 16)]
                    s16 = srcp_v[pl.ds(gg * 16, 16)]
                    m = (d16 >= lo) & (d16 < lo + ch)
                    mi = m.astype(jnp.int32)
                    wr = cnt + plsc.cumsum(mi) - mi
                    plsc.store_scatter(pend_d, [wr], d16 - lo, mask=m)
                    plsc.store_scatter(pend_s, [wr], s16, mask=m)
                    if with_deg:
                        plsc.addupdate_scatter(
                            deg_v, [jnp.where(m, d16 - lo, 0)], ones16,
                            mask=m)
                    return cnt + jnp.sum(mi)
                cnt = lax.fori_loop(0, NGRP, scan_g, jnp.int32(0))

                nbat = (cnt + (g - 1)) // g

                def bat(b, _):
                    ds16, vals = [], []
                    for j in range(g // 16):
                        s16 = pend_s[pl.ds(b * g + j * 16, 16)]
                        d16 = pend_d[pl.ds(b * g + j * 16, 16)]
                        valid = (b * g + j * 16 + iota16) < cnt
                        isrc_v[pl.ds(j * 16, 16)] = jnp.where(valid, s16, 0)
                        ds16.append(jnp.where(valid, d16, 0))
                        vals.append(valid)
                    pltpu.async_copy(x_hbm.at[isrc_v], rows_v, sem).wait()
                    for j in range(g // 16):
                        rbase = j * 16 + iota16

                        def colgrp(cg, _):
                            for cc in range(16):
                                col = cg * 16 + cc
                                cv = plsc.load_gather(
                                    rows_v, [rbase, jnp.full(
                                        (16,), col, jnp.int32)])
                                plsc.addupdate_scatter(
                                    acc_v,
                                    [ds16[j],
                                     jnp.full((16,), col, jnp.int32)],
                                    cv, mask=vals[j])
                            return 0
                        lax.fori_loop(0, d // 16, colgrp, 0)
                    return 0
                lax.fori_loop(0, nbat, bat, 0)
                return 0
            lax.fori_loop(0, NPIECE, piece, 0)

            @pl.when(lo < N_PAD)
            def _():
                pltpu.sync_copy(acc_v, agg_hbm.at[pl.ds(cbase + lo, ch)])
                if with_deg:
                    pltpu.sync_copy(deg_v, deg_hbm.at[pl.ds(cbase + lo, ch)])

    mesh = plsc.VectorSubcoreMesh(core_axis_name="c", subcore_axis_name="s",
                                  num_cores=NSC, num_subcores=NTILES)
    return pl.kernel(body, out_type=tuple(outs) if with_deg else outs[0],
                     mesh=mesh, scratch_types=scratch,
                     compiler_params=pltpu.CompilerParams(
                         needs_layout_passes=False))


BM = 1024


def _tc_layer_body(a0_ref, a1_ref, d0_ref, d1_ref, x_ref, wl_ref, wr_ref,
                   b_ref, o_ref, *, relu):
    deg = d0_ref[...] + d1_ref[...]
    mean = (a0_ref[...] + a1_ref[...]) / jnp.maximum(deg, 1.0)
    acc = jnp.dot(mean, wl_ref[...], preferred_element_type=jnp.float32)
    acc += jnp.dot(x_ref[...], wr_ref[...], preferred_element_type=jnp.float32)
    acc += b_ref[...]
    o_ref[...] = jnp.maximum(acc, 0.0) if relu else acc


def _tc_layer(a0, a1, d0, d1, x, wlT, wrT, b2d, relu):
    din, dout = wlT.shape
    grid = N_PAD // BM
    return pl.pallas_call(
        functools.partial(_tc_layer_body, relu=relu),
        grid=(grid,),
        in_specs=[
            pl.BlockSpec((BM, din), lambda i: (i, 0)),
            pl.BlockSpec((BM, din), lambda i: (i, 0)),
            pl.BlockSpec((BM, 1), lambda i: (i, 0)),
            pl.BlockSpec((BM, 1), lambda i: (i, 0)),
            pl.BlockSpec((BM, din), lambda i: (i, 0)),
            pl.BlockSpec((din, dout), lambda i: (0, 0)),
            pl.BlockSpec((din, dout), lambda i: (0, 0)),
            pl.BlockSpec((1, dout), lambda i: (0, 0)),
        ],
        out_specs=pl.BlockSpec((BM, dout), lambda i: (i, 0)),
        out_shape=jax.ShapeDtypeStruct((N_PAD, dout), jnp.float32),
    )(a0, a1, d0, d1, x, wlT, wrT, b2d)


def _tc_dual_mm_body(x_ref, wl_ref, wr_ref, b_ref, p_ref, q_ref):
    xv = x_ref[...]
    p_ref[...] = jnp.dot(xv, wl_ref[...], preferred_element_type=jnp.float32)
    q_ref[...] = jnp.dot(xv, wr_ref[...],
                         preferred_element_type=jnp.float32) + b_ref[...]


def _tc_dual_mm(x, wlT, wrT, b2d):
    din, dout = wlT.shape
    grid = N_PAD // BM
    return pl.pallas_call(
        _tc_dual_mm_body,
        grid=(grid,),
        in_specs=[
            pl.BlockSpec((BM, din), lambda i: (i, 0)),
            pl.BlockSpec((din, dout), lambda i: (0, 0)),
            pl.BlockSpec((din, dout), lambda i: (0, 0)),
            pl.BlockSpec((1, dout), lambda i: (0, 0)),
        ],
        out_specs=[pl.BlockSpec((BM, dout), lambda i: (i, 0)),
                   pl.BlockSpec((BM, dout), lambda i: (i, 0))],
        out_shape=[jax.ShapeDtypeStruct((N_PAD, dout), jnp.float32),
                   jax.ShapeDtypeStruct((N_PAD, dout), jnp.float32)],
    )(x, wlT, wrT, b2d)


def _tc_combine_body(a0_ref, a1_ref, d0_ref, d1_ref, q_ref, o_ref):
    deg = d0_ref[...] + d1_ref[...]
    o_ref[...] = (a0_ref[...] + a1_ref[...]) / jnp.maximum(deg, 1.0) \
        + q_ref[...]


def _tc_combine(a0, a1, d0, d1, q):
    dout = a0.shape[1]
    grid = N_PAD // BM
    return pl.pallas_call(
        _tc_combine_body,
        grid=(grid,),
        in_specs=[
            pl.BlockSpec((BM, dout), lambda i: (i, 0)),
            pl.BlockSpec((BM, dout), lambda i: (i, 0)),
            pl.BlockSpec((BM, 1), lambda i: (i, 0)),
            pl.BlockSpec((BM, 1), lambda i: (i, 0)),
            pl.BlockSpec((BM, dout), lambda i: (i, 0)),
        ],
        out_specs=pl.BlockSpec((BM, dout), lambda i: (i, 0)),
        out_shape=jax.ShapeDtypeStruct((N_PAD, dout), jnp.float32),
    )(a0, a1, d0, d1, q)


_make_sc_agg_cached = functools.lru_cache(maxsize=None)(_make_sc_agg)


@jax.jit
def kernel(x, edge_index, W1l, W1r, b1, W2l, W2r, b2, W3l, W3r, b3):
    npad_e = E_PAD - N_EDGES
    src = jnp.concatenate([edge_index[0],
                           jnp.zeros((npad_e,), jnp.int32)])
    dst = jnp.concatenate([edge_index[1],
                           N_NODES + (jnp.arange(npad_e, dtype=jnp.int32)
                                      % (N_PAD - N_NODES))])
    xp = jnp.pad(x, ((0, N_PAD - N_NODES), (0, 0)))

    agg1, deg = _make_sc_agg_cached(256, True)(xp, src, dst)
    a0, a1 = agg1[:N_PAD], agg1[N_PAD:]
    d0, d1 = deg[:N_PAD, None], deg[N_PAD:, None]
    h1 = _tc_layer(a0, a1, d0, d1, xp, W1l.T, W1r.T, b1[None, :], relu=True)

    agg2 = _make_sc_agg_cached(512, False)(h1, src, dst)
    h2 = _tc_layer(agg2[:N_PAD], agg2[N_PAD:], d0, d1, h1,
                   W2l.T, W2r.T, b2[None, :], relu=True)

    p, q = _tc_dual_mm(h2, W3l.T, W3r.T, b3[None, :])
    agg3 = _make_sc_agg_cached(256, False)(p, src, dst)
    out = _tc_combine(agg3[:N_PAD], agg3[N_PAD:], d0, d1, q)
    return out[:N_NODES]


# speedup vs baseline: 1.5765x; 1.5765x over previous
import functools

import jax
import jax.numpy as jnp
from jax import lax
from jax.experimental import pallas as pl
from jax.experimental.pallas import tpu as pltpu
from jax.experimental.pallas import tpu_sc as plsc

N_NODES = 10000
N_EDGES = 160000
N_PAD = 10240
NSC = 2
NTILES = 16
E_PAD = 160256
HALF_E = E_PAD // NSC
NPIECE = 16
PC = HALF_E // NPIECE
NGRP = PC // 16
PCAP = PC + 16


def _make_sc_agg(d, with_deg):
    ch = 256 if d == 256 else 128
    g = 80 if d == 256 else 48
    nch = -(-N_PAD // ch)
    npass = -(-nch // NTILES)

    outs = [jax.ShapeDtypeStruct((NSC * N_PAD, d), jnp.float32)]
    if with_deg:
        outs.append(jax.ShapeDtypeStruct((NSC * N_PAD,), jnp.float32))

    scratch = [
        pltpu.VMEM((PC,), jnp.int32),
        pltpu.VMEM((PC,), jnp.int32),
        pltpu.VMEM((PCAP,), jnp.int32),
        pltpu.VMEM((PCAP,), jnp.int32),
        pltpu.VMEM((g, d), jnp.float32),
        pltpu.VMEM((g,), jnp.int32),
        pltpu.VMEM((ch, d), jnp.float32),
        pltpu.SemaphoreType.DMA,
    ]
    if with_deg:
        scratch += [pltpu.VMEM((ch,), jnp.float32)]

    def body(x_hbm, src_hbm, dst_hbm, *refs):
        if with_deg:
            (agg_hbm, deg_hbm, srcp_v, dstp_v, pend_s, pend_d,
             rows_v, isrc_v, acc_v, sem, deg_v) = refs
        else:
            (agg_hbm, srcp_v, dstp_v, pend_s, pend_d,
             rows_v, isrc_v, acc_v, sem) = refs

        c = lax.axis_index("c")
        s = lax.axis_index("s")
        cbase = c * N_PAD
        z16 = jnp.zeros((16,), jnp.float32)
        zi16 = jnp.zeros((16,), jnp.int32)
        ones16 = jnp.full((16,), 1.0, jnp.float32)
        iota16 = lax.iota(jnp.int32, 16)

        def pzbody(i, _):
            pend_s[pl.ds(i * 16, 16)] = zi16
            pend_d[pl.ds(i * 16, 16)] = zi16
            return 0
        lax.fori_loop(0, PCAP // 16, pzbody, 0)

        for ps in range(npass):
            q = ps * NTILES + s
            lo = q * ch

            def azbody(r, _):
                for cc in range(d // 16):
                    acc_v[r, pl.ds(cc * 16, 16)] = z16
                return 0
            lax.fori_loop(0, ch, azbody, 0)
            if with_deg:
                def dzbody(i, _):
                    deg_v[pl.ds(i * 16, 16)] = z16
                    return 0
                lax.fori_loop(0, ch // 16, dzbody, 0)

            def piece(p, _):
                ebase = c * HALF_E + p * PC
                pltpu.sync_copy(src_hbm.at[pl.ds(ebase, PC)], srcp_v)
                pltpu.sync_copy(dst_hbm.at[pl.ds(ebase, PC)], dstp_v)

                def scan_g(gg, cnt):
                    d16 = dstp_v[pl.ds(gg * 16, 16)]
                    s16 = srcp_v[pl.ds(gg * 16, 16)]
                    m = (d16 >= lo) & (d16 < lo + ch)
                    mi = m.astype(jnp.int32)
                    wr = cnt + plsc.cumsum(mi) - mi
                    plsc.store_scatter(pend_d, [wr], d16 - lo, mask=m)
                    plsc.store_scatter(pend_s, [wr], s16, mask=m)
                    if with_deg:
                        plsc.addupdate_scatter(
                            deg_v, [jnp.where(m, d16 - lo, 0)], ones16,
                            mask=m)
                    return cnt + jnp.sum(mi)
                cnt = lax.fori_loop(0, NGRP, scan_g, jnp.int32(0))

                nbat = (cnt + (g - 1)) // g

                def bat(b, _):
                    for j in range(g // 16):
                        s16 = pend_s[pl.ds(b * g + j * 16, 16)]
                        valid = (b * g + j * 16 + iota16) < cnt
                        isrc_v[pl.ds(j * 16, 16)] = jnp.where(valid, s16, 0)
                    pltpu.async_copy(x_hbm.at[isrc_v], rows_v, sem).wait()

                    def rowadd(r, _):
                        grp = r // 16
                        d16 = pend_d[pl.ds(b * g + grp * 16, 16)]
                        dloc = jnp.sum(jnp.where(iota16 == (r - grp * 16),
                                                 d16, 0))

                        @pl.when(b * g + r < cnt)
                        def _():
                            for cc in range(d // 16):
                                sl = pl.ds(cc * 16, 16)
                                acc_v[dloc, sl] = (acc_v[dloc, sl]
                                                   + rows_v[r, sl])
                        return 0
                    lax.fori_loop(0, g, rowadd, 0)
                    return 0
                lax.fori_loop(0, nbat, bat, 0)
                return 0
            lax.fori_loop(0, NPIECE, piece, 0)

            @pl.when(lo < N_PAD)
            def _():
                pltpu.sync_copy(acc_v, agg_hbm.at[pl.ds(cbase + lo, ch)])
                if with_deg:
                    pltpu.sync_copy(deg_v, deg_hbm.at[pl.ds(cbase + lo, ch)])

    mesh = plsc.VectorSubcoreMesh(core_axis_name="c", subcore_axis_name="s",
                                  num_cores=NSC, num_subcores=NTILES)
    return pl.kernel(body, out_type=tuple(outs) if with_deg else outs[0],
                     mesh=mesh, scratch_types=scratch,
                     compiler_params=pltpu.CompilerParams(
                         needs_layout_passes=False))


BM = 1024


def _tc_layer_body(a0_ref, a1_ref, d0_ref, d1_ref, x_ref, wl_ref, wr_ref,
                   b_ref, o_ref, *, relu):
    deg = d0_ref[...] + d1_ref[...]
    mean = (a0_ref[...] + a1_ref[...]) / jnp.maximum(deg, 1.0)
    acc = jnp.dot(mean, wl_ref[...], preferred_element_type=jnp.float32)
    acc += jnp.dot(x_ref[...], wr_ref[...], preferred_element_type=jnp.float32)
    acc += b_ref[...]
    o_ref[...] = jnp.maximum(acc, 0.0) if relu else acc


def _tc_layer(a0, a1, d0, d1, x, wlT, wrT, b2d, relu):
    din, dout = wlT.shape
    grid = N_PAD // BM
    return pl.pallas_call(
        functools.partial(_tc_layer_body, relu=relu),
        grid=(grid,),
        in_specs=[
            pl.BlockSpec((BM, din), lambda i: (i, 0)),
            pl.BlockSpec((BM, din), lambda i: (i, 0)),
            pl.BlockSpec((BM, 1), lambda i: (i, 0)),
            pl.BlockSpec((BM, 1), lambda i: (i, 0)),
            pl.BlockSpec((BM, din), lambda i: (i, 0)),
            pl.BlockSpec((din, dout), lambda i: (0, 0)),
            pl.BlockSpec((din, dout), lambda i: (0, 0)),
            pl.BlockSpec((1, dout), lambda i: (0, 0)),
        ],
        out_specs=pl.BlockSpec((BM, dout), lambda i: (i, 0)),
        out_shape=jax.ShapeDtypeStruct((N_PAD, dout), jnp.float32),
    )(a0, a1, d0, d1, x, wlT, wrT, b2d)


def _tc_dual_mm_body(x_ref, wl_ref, wr_ref, b_ref, p_ref, q_ref):
    xv = x_ref[...]
    p_ref[...] = jnp.dot(xv, wl_ref[...], preferred_element_type=jnp.float32)
    q_ref[...] = jnp.dot(xv, wr_ref[...],
                         preferred_element_type=jnp.float32) + b_ref[...]


def _tc_dual_mm(x, wlT, wrT, b2d):
    din, dout = wlT.shape
    grid = N_PAD // BM
    return pl.pallas_call(
        _tc_dual_mm_body,
        grid=(grid,),
        in_specs=[
            pl.BlockSpec((BM, din), lambda i: (i, 0)),
            pl.BlockSpec((din, dout), lambda i: (0, 0)),
            pl.BlockSpec((din, dout), lambda i: (0, 0)),
            pl.BlockSpec((1, dout), lambda i: (0, 0)),
        ],
        out_specs=[pl.BlockSpec((BM, dout), lambda i: (i, 0)),
                   pl.BlockSpec((BM, dout), lambda i: (i, 0))],
        out_shape=[jax.ShapeDtypeStruct((N_PAD, dout), jnp.float32),
                   jax.ShapeDtypeStruct((N_PAD, dout), jnp.float32)],
    )(x, wlT, wrT, b2d)


def _tc_combine_body(a0_ref, a1_ref, d0_ref, d1_ref, q_ref, o_ref):
    deg = d0_ref[...] + d1_ref[...]
    o_ref[...] = (a0_ref[...] + a1_ref[...]) / jnp.maximum(deg, 1.0) \
        + q_ref[...]


def _tc_combine(a0, a1, d0, d1, q):
    dout = a0.shape[1]
    grid = N_PAD // BM
    return pl.pallas_call(
        _tc_combine_body,
        grid=(grid,),
        in_specs=[
            pl.BlockSpec((BM, dout), lambda i: (i, 0)),
            pl.BlockSpec((BM, dout), lambda i: (i, 0)),
            pl.BlockSpec((BM, 1), lambda i: (i, 0)),
            pl.BlockSpec((BM, 1), lambda i: (i, 0)),
            pl.BlockSpec((BM, dout), lambda i: (i, 0)),
        ],
        out_specs=pl.BlockSpec((BM, dout), lambda i: (i, 0)),
        out_shape=jax.ShapeDtypeStruct((N_PAD, dout), jnp.float32),
    )(a0, a1, d0, d1, q)


_make_sc_agg_cached = functools.lru_cache(maxsize=None)(_make_sc_agg)


@jax.jit
def kernel(x, edge_index, W1l, W1r, b1, W2l, W2r, b2, W3l, W3r, b3):
    npad_e = E_PAD - N_EDGES
    src = jnp.concatenate([edge_index[0],
                           jnp.zeros((npad_e,), jnp.int32)])
    dst = jnp.concatenate([edge_index[1],
                           N_NODES + (jnp.arange(npad_e, dtype=jnp.int32)
                                      % (N_PAD - N_NODES))])
    xp = jnp.pad(x, ((0, N_PAD - N_NODES), (0, 0)))

    agg1, deg = _make_sc_agg_cached(256, True)(xp, src, dst)
    a0, a1 = agg1[:N_PAD], agg1[N_PAD:]
    d0, d1 = deg[:N_PAD, None], deg[N_PAD:, None]
    h1 = _tc_layer(a0, a1, d0, d1, xp, W1l.T, W1r.T, b1[None, :], relu=True)

    agg2 = _make_sc_agg_cached(512, False)(h1, src, dst)
    h2 = _tc_layer(agg2[:N_PAD], agg2[N_PAD:], d0, d1, h1,
                   W2l.T, W2r.T, b2[None, :], relu=True)

    p, q = _tc_dual_mm(h2, W3l.T, W3r.T, b3[None, :])
    agg3 = _make_sc_agg_cached(256, False)(p, src, dst)
    out = _tc_combine(agg3[:N_PAD], agg3[N_PAD:], d0, d1, q)
    return out[:N_NODES]


# speedup vs baseline: 1.6024x; 1.0164x over previous
import functools

import jax
import jax.numpy as jnp
from jax import lax
from jax.experimental import pallas as pl
from jax.experimental.pallas import tpu as pltpu
from jax.experimental.pallas import tpu_sc as plsc

N_NODES = 10000
N_EDGES = 160000
N_PAD = 10240
NSC = 2
NTILES = 16
E_PAD = 160256
HALF_E = E_PAD // NSC
NPIECE = 16
PC = HALF_E // NPIECE
NGRP = PC // 16
PCAP = PC + 16


def _make_sc_agg(d, with_deg):
    ch = 256 if d == 256 else 128
    g = 80 if d == 256 else 48
    nch = -(-N_PAD // ch)
    npass = -(-nch // NTILES)

    outs = [jax.ShapeDtypeStruct((NSC * N_PAD, d), jnp.float32)]
    if with_deg:
        outs.append(jax.ShapeDtypeStruct((NSC * N_PAD,), jnp.float32))

    scratch = [
        pltpu.VMEM((PC,), jnp.int32),
        pltpu.VMEM((PC,), jnp.int32),
        pltpu.VMEM((PCAP,), jnp.int32),
        pltpu.VMEM((PCAP,), jnp.int32),
        pltpu.VMEM((g, d), jnp.float32),
        pltpu.VMEM((g,), jnp.int32),
        pltpu.VMEM((ch, d), jnp.float32),
        pltpu.SemaphoreType.DMA,
    ]
    if with_deg:
        scratch += [pltpu.VMEM((ch,), jnp.float32)]

    def body(x_hbm, src_hbm, dst_hbm, *refs):
        if with_deg:
            (agg_hbm, deg_hbm, srcp_v, dstp_v, pend_s, pend_d,
             rows_v, isrc_v, acc_v, sem, deg_v) = refs
        else:
            (agg_hbm, srcp_v, dstp_v, pend_s, pend_d,
             rows_v, isrc_v, acc_v, sem) = refs

        c = lax.axis_index("c")
        s = lax.axis_index("s")
        cbase = c * N_PAD
        z16 = jnp.zeros((16,), jnp.float32)
        zi16 = jnp.zeros((16,), jnp.int32)
        ones16 = jnp.full((16,), 1.0, jnp.float32)
        iota16 = lax.iota(jnp.int32, 16)

        def pzbody(i, _):
            pend_s[pl.ds(i * 16, 16)] = zi16
            pend_d[pl.ds(i * 16, 16)] = zi16
            return 0
        lax.fori_loop(0, PCAP // 16, pzbody, 0)

        for ps in range(npass):
            q = ps * NTILES + s
            lo = q * ch

            def azbody(r, _):
                for cc in range(d // 16):
                    acc_v[r, pl.ds(cc * 16, 16)] = z16
                return 0
            lax.fori_loop(0, ch, azbody, 0)
            if with_deg:
                def dzbody(i, _):
                    deg_v[pl.ds(i * 16, 16)] = z16
                    return 0
                lax.fori_loop(0, ch // 16, dzbody, 0)

            def piece(p, _):
                ebase = c * HALF_E + p * PC
                pltpu.sync_copy(src_hbm.at[pl.ds(ebase, PC)], srcp_v)
                pltpu.sync_copy(dst_hbm.at[pl.ds(ebase, PC)], dstp_v)

                def scan_g(gg, cntv):
                    d16 = dstp_v[pl.ds(gg * 16, 16)]
                    s16 = srcp_v[pl.ds(gg * 16, 16)]
                    m = (d16 >= lo) & (d16 < lo + ch)
                    mi = m.astype(jnp.int32)
                    wr = cntv + plsc.cumsum(mi) - mi
                    plsc.store_scatter(pend_d, [wr], d16 - lo, mask=m)
                    plsc.store_scatter(pend_s, [wr], s16, mask=m)
                    if with_deg:
                        plsc.addupdate_scatter(
                            deg_v, [jnp.where(m, d16 - lo, 0)], ones16,
                            mask=m)
                    return cntv + plsc.all_reduce_population_count(m)
                cntv = lax.fori_loop(0, NGRP, scan_g,
                                     jnp.zeros((16,), jnp.int32))
                cnt = jnp.max(cntv)

                nbat = (cnt + (g - 1)) // g

                def bat(b, _):
                    for j in range(g // 16):
                        s16 = pend_s[pl.ds(b * g + j * 16, 16)]
                        valid = (b * g + j * 16 + iota16) < cnt
                        isrc_v[pl.ds(j * 16, 16)] = jnp.where(valid, s16, 0)
                    pltpu.async_copy(x_hbm.at[isrc_v], rows_v, sem).wait()

                    def rowadd(r, _):
                        dsp = plsc.load_gather(
                            pend_d, [jnp.full((16,), b * g + r, jnp.int32)])

                        @pl.when(b * g + r < cnt)
                        def _():
                            for cc in range(d // 16):
                                plsc.addupdate_scatter(
                                    acc_v, [dsp, cc * 16 + iota16],
                                    rows_v[r, pl.ds(cc * 16, 16)])
                        return 0
                    lax.fori_loop(0, g, rowadd, 0)
                    return 0
                lax.fori_loop(0, nbat, bat, 0)
                return 0
            lax.fori_loop(0, NPIECE, piece, 0)

            @pl.when(lo < N_PAD)
            def _():
                pltpu.sync_copy(acc_v, agg_hbm.at[pl.ds(cbase + lo, ch)])
                if with_deg:
                    pltpu.sync_copy(deg_v, deg_hbm.at[pl.ds(cbase + lo, ch)])

    mesh = plsc.VectorSubcoreMesh(core_axis_name="c", subcore_axis_name="s",
                                  num_cores=NSC, num_subcores=NTILES)
    return pl.kernel(body, out_type=tuple(outs) if with_deg else outs[0],
                     mesh=mesh, scratch_types=scratch,
                     compiler_params=pltpu.CompilerParams(
                         needs_layout_passes=False))


BM = 1024


def _tc_layer_body(a0_ref, a1_ref, d0_ref, d1_ref, x_ref, wl_ref, wr_ref,
                   b_ref, o_ref, *, relu):
    deg = d0_ref[...] + d1_ref[...]
    mean = (a0_ref[...] + a1_ref[...]) / jnp.maximum(deg, 1.0)
    acc = jnp.dot(mean, wl_ref[...], preferred_element_type=jnp.float32)
    acc += jnp.dot(x_ref[...], wr_ref[...], preferred_element_type=jnp.float32)
    acc += b_ref[...]
    o_ref[...] = jnp.maximum(acc, 0.0) if relu else acc


def _tc_layer(a0, a1, d0, d1, x, wlT, wrT, b2d, relu):
    din, dout = wlT.shape
    grid = N_PAD // BM
    return pl.pallas_call(
        functools.partial(_tc_layer_body, relu=relu),
        grid=(grid,),
        in_specs=[
            pl.BlockSpec((BM, din), lambda i: (i, 0)),
            pl.BlockSpec((BM, din), lambda i: (i, 0)),
            pl.BlockSpec((BM, 1), lambda i: (i, 0)),
            pl.BlockSpec((BM, 1), lambda i: (i, 0)),
            pl.BlockSpec((BM, din), lambda i: (i, 0)),
            pl.BlockSpec((din, dout), lambda i: (0, 0)),
            pl.BlockSpec((din, dout), lambda i: (0, 0)),
            pl.BlockSpec((1, dout), lambda i: (0, 0)),
        ],
        out_specs=pl.BlockSpec((BM, dout), lambda i: (i, 0)),
        out_shape=jax.ShapeDtypeStruct((N_PAD, dout), jnp.float32),
    )(a0, a1, d0, d1, x, wlT, wrT, b2d)


def _tc_dual_mm_body(x_ref, wl_ref, wr_ref, b_ref, p_ref, q_ref):
    xv = x_ref[...]
    p_ref[...] = jnp.dot(xv, wl_ref[...], preferred_element_type=jnp.float32)
    q_ref[...] = jnp.dot(xv, wr_ref[...],
                         preferred_element_type=jnp.float32) + b_ref[...]


def _tc_dual_mm(x, wlT, wrT, b2d):
    din, dout = wlT.shape
    grid = N_PAD // BM
    return pl.pallas_call(
        _tc_dual_mm_body,
        grid=(grid,),
        in_specs=[
            pl.BlockSpec((BM, din), lambda i: (i, 0)),
            pl.BlockSpec((din, dout), lambda i: (0, 0)),
            pl.BlockSpec((din, dout), lambda i: (0, 0)),
            pl.BlockSpec((1, dout), lambda i: (0, 0)),
        ],
        out_specs=[pl.BlockSpec((BM, dout), lambda i: (i, 0)),
                   pl.BlockSpec((BM, dout), lambda i: (i, 0))],
        out_shape=[jax.ShapeDtypeStruct((N_PAD, dout), jnp.float32),
                   jax.ShapeDtypeStruct((N_PAD, dout), jnp.float32)],
    )(x, wlT, wrT, b2d)


def _tc_combine_body(a0_ref, a1_ref, d0_ref, d1_ref, q_ref, o_ref):
    deg = d0_ref[...] + d1_ref[...]
    o_ref[...] = (a0_ref[...] + a1_ref[...]) / jnp.maximum(deg, 1.0) \
        + q_ref[...]


def _tc_combine(a0, a1, d0, d1, q):
    dout = a0.shape[1]
    grid = N_PAD // BM
    return pl.pallas_call(
        _tc_combine_body,
        grid=(grid,),
        in_specs=[
            pl.BlockSpec((BM, dout), lambda i: (i, 0)),
            pl.BlockSpec((BM, dout), lambda i: (i, 0)),
            pl.BlockSpec((BM, 1), lambda i: (i, 0)),
            pl.BlockSpec((BM, 1), lambda i: (i, 0)),
            pl.BlockSpec((BM, dout), lambda i: (i, 0)),
        ],
        out_specs=pl.BlockSpec((BM, dout), lambda i: (i, 0)),
        out_shape=jax.ShapeDtypeStruct((N_PAD, dout), jnp.float32),
    )(a0, a1, d0, d1, q)


_make_sc_agg_cached = functools.lru_cache(maxsize=None)(_make_sc_agg)


@jax.jit
def kernel(x, edge_index, W1l, W1r, b1, W2l, W2r, b2, W3l, W3r, b3):
    npad_e = E_PAD - N_EDGES
    src = jnp.concatenate([edge_index[0],
                           jnp.zeros((npad_e,), jnp.int32)])
    dst = jnp.concatenate([edge_index[1],
                           N_NODES + (jnp.arange(npad_e, dtype=jnp.int32)
                                      % (N_PAD - N_NODES))])
    xp = jnp.pad(x, ((0, N_PAD - N_NODES), (0, 0)))

    agg1, deg = _make_sc_agg_cached(256, True)(xp, src, dst)
    a0, a1 = agg1[:N_PAD], agg1[N_PAD:]
    d0, d1 = deg[:N_PAD, None], deg[N_PAD:, None]
    h1 = _tc_layer(a0, a1, d0, d1, xp, W1l.T, W1r.T, b1[None, :], relu=True)

    agg2 = _make_sc_agg_cached(512, False)(h1, src, dst)
    h2 = _tc_layer(agg2[:N_PAD], agg2[N_PAD:], d0, d1, h1,
                   W2l.T, W2r.T, b2[None, :], relu=True)

    p, q = _tc_dual_mm(h2, W3l.T, W3r.T, b3[None, :])
    agg3 = _make_sc_agg_cached(256, False)(p, src, dst)
    out = _tc_combine(agg3[:N_PAD], agg3[N_PAD:], d0, d1, q)
    return out[:N_NODES]
